# baseline (speedup 1.0000x reference)
import functools

import jax
import jax.numpy as jnp
from jax import lax
from jax.experimental import pallas as pl
from jax.experimental.pallas import tpu as pltpu
from jax.experimental.pallas import tpu_sc as plsc

N = 10000
E = 320000
G = 256
D = 128
DH = 64
NC = 2
NS = 16
EPS = E // NS
CH = 400
NCHUNK = EPS // CH
ROWS_PS = N // NS

_POOL_R = 1024
_N_PAD = 10240


NH = N // 2
NHA = 5008
EC = 163840
EPSUB = EC // NS
CHE = 320
NCHE = EPSUB // CHE


def _sc_agg_body(xe_ref, eidx_ref, src_ref, dstl_ref, xn_ref, part_ref,
                 acc, eidx_v, src_v, dst_v, zbuf_v, data_v, tmp_v, gsem, esem):
    i32 = jnp.int32
    c = lax.axis_index("c").astype(i32)
    s = lax.axis_index("s").astype(i32)

    def _zero(i, _):
        for j in range(D // 16):
            zbuf_v[i, pl.ds(j * 16, 16)] = jnp.zeros((16,), jnp.float32)
        return 0
    lax.fori_loop(jnp.int32(0), jnp.int32(32), _zero, 0)
    z0 = s * i32(313)
    for off in range(0, 288, 32):
        pltpu.sync_copy(zbuf_v, acc.at[pl.ds(z0 + i32(off), 32)])
    pltpu.sync_copy(zbuf_v.at[pl.ds(0, 25)], acc.at[pl.ds(z0 + i32(288), 25)])
    plsc.subcore_barrier()

    def _chunk(k, _):
        base = pl.multiple_of(c * i32(EC) + s * i32(EPSUB) + k * i32(CHE), 8)
        pltpu.sync_copy(eidx_ref.at[pl.ds(base, CHE)], eidx_v)
        pltpu.sync_copy(src_ref.at[pl.ds(base, CHE)], src_v)
        pltpu.sync_copy(dstl_ref.at[pl.ds(base, CHE)], dst_v)
        cp_xe = pltpu.async_copy(xe_ref.at[eidx_v], data_v, esem)
        cp_xn = pltpu.async_copy(xn_ref.at[src_v], tmp_v, gsem)
        cp_xe.wait()
        pltpu.sync_copy(data_v, acc.at[dst_v], add=True)
        cp_xn.wait()
        pltpu.sync_copy(tmp_v, acc.at[dst_v], add=True)
        return 0
    lax.fori_loop(jnp.int32(0), jnp.int32(NCHE), _chunk, 0)

    plsc.subcore_barrier()

    @pl.when(s < 12)
    def _writeout():
        r0 = pl.multiple_of(s * i32(416), 8)
        pltpu.sync_copy(acc.at[pl.ds(r0, 416)],
                        part_ref.at[c, pl.ds(r0, 416)])

    @pl.when(s == 12)
    def _writeout_tail():
        pltpu.sync_copy(acc.at[pl.ds(4992, 8)],
                        part_ref.at[c, pl.ds(i32(4992), 8)])


def _partition_edges(src, dst):
    side = (dst >= NH).astype(jnp.int32)
    r1 = jnp.cumsum(side) - side
    r0 = jnp.cumsum(1 - side) - (1 - side)
    pos = jnp.where(side == 1, EC + r1, r0)
    e_iota = jnp.arange(E, dtype=jnp.int32)
    eidx = jnp.zeros((2 * EC,), jnp.int32).at[pos].set(e_iota)
    src_p = jnp.zeros((2 * EC,), jnp.int32).at[pos].set(src)
    dst_l = jnp.full((2 * EC,), NH, jnp.int32).at[pos].set(dst - NH * side)
    return eidx, src_p, dst_l


def _sc_agg(xe, eidx, src_p, dst_l, xn):
    mesh = plsc.VectorSubcoreMesh(core_axis_name="c", subcore_axis_name="s")
    f = pl.kernel(
        _sc_agg_body,
        out_type=jax.ShapeDtypeStruct((NC, NH, D), jnp.float32),
        mesh=mesh,
        scratch_types=[
            pltpu.VMEM_SHARED((NHA, D), jnp.float32),
            pltpu.VMEM((CHE,), jnp.int32),
            pltpu.VMEM((CHE,), jnp.int32),
            pltpu.VMEM((CHE,), jnp.int32),
            pltpu.VMEM((32, D), jnp.float32),
            pltpu.VMEM((CHE, D), jnp.float32),
            pltpu.VMEM((CHE, D), jnp.float32),
            pltpu.SemaphoreType.DMA,
            pltpu.SemaphoreType.DMA,
        ],
    )
    return f(xe, eidx, src_p, dst_l, xn)


def _pool_body(idx_ref, x_ref, o_ref, acc, cnt):
    step = pl.program_id(0)

    @pl.when(step == 0)
    def _init():
        acc[...] = jnp.zeros_like(acc)
        cnt[...] = jnp.zeros_like(cnt)

    idx = idx_ref[0, 0, :]
    onehot = (jax.lax.broadcasted_iota(jnp.int32, (G, _POOL_R), 0)
              == idx[None, :]).astype(jnp.float32)
    acc[...] += jnp.dot(onehot, x_ref[...],
                        preferred_element_type=jnp.float32)
    cnt[...] += jnp.sum(onehot, axis=1, keepdims=True)

    @pl.when(step == pl.num_programs(0) - 1)
    def _fini():
        o_ref[...] = acc[...] / jnp.clip(cnt[...], 1.0, None)


def _pool(x, idx):
    xp = jnp.zeros((_N_PAD, D), jnp.float32).at[:N].set(x)
    ip = jnp.full((_N_PAD,), G, jnp.int32).at[:N].set(idx.astype(jnp.int32))
    ip = ip.reshape(_N_PAD // _POOL_R, 1, _POOL_R)
    grid = _N_PAD // _POOL_R
    return pl.pallas_call(
        _pool_body,
        grid=(grid,),
        in_specs=[
            pl.BlockSpec((1, 1, _POOL_R), lambda i: (i, i * 0, i * 0)),
            pl.BlockSpec((_POOL_R, D), lambda i: (i, i * 0)),
        ],
        out_specs=pl.BlockSpec((G, D), lambda i: (i * 0, i * 0)),
        out_shape=jax.ShapeDtypeStruct((G, D), jnp.float32),
        scratch_shapes=[
            pltpu.VMEM((G, D), jnp.float32),
            pltpu.VMEM((G, 1), jnp.float32),
        ],
    )(ip, xp)


@functools.partial(jax.jit, static_argnums=())
def kernel(node, edge, n2n, e2n, idx_node, idx_edge, W_en, b_en, W_ee, b_ee,
           Wn, bn, We, be, W_fc, b_fc):
    del n2n, idx_edge, W_fc, b_fc
    src = e2n[0].astype(jnp.int32)
    dst = e2n[1].astype(jnp.int32)
    xn = node @ W_en + b_en
    xe = edge @ W_ee + b_ee
    L = Wn.shape[0]
    eidx, src_p, dst_l = _partition_edges(src, dst)
    for i in range(L):
        part = _sc_agg(xe, eidx, src_p, dst_l, xn)
        agg = part.reshape(N, D)
        xn = jax.nn.relu((xn + agg) @ Wn[i] + bn[i])
        if i < L - 1:
            xe = jax.nn.relu(
                (xe + jnp.take(xn, src, axis=0) + jnp.take(xn, dst, axis=0))
                @ We[i] + be[i])
    return _pool(xn, idx_node)


# speedup vs baseline: 2.1097x; 2.1097x over previous
import functools

import jax
import jax.numpy as jnp
from jax import lax
from jax.experimental import pallas as pl
from jax.experimental.pallas import tpu as pltpu
from jax.experimental.pallas import tpu_sc as plsc

N = 10000
E = 320000
G = 256
D = 128
DH = 64
NC = 2
NS = 16
EPS = E // NS
CH = 400
NCHUNK = EPS // CH
ROWS_PS = N // NS

_POOL_R = 1024
_N_PAD = 10240


EPC = E // NC
EPSUB = EPC // NS
CHE = 80
NCHE = EPSUB // CHE


def _sc_agg_body(xe_ref, src_ref, dst_ref, xn_ref, part_ref,
                 acc, src_v0, src_v1, dst_v0, dst_v1, eidx_v0, eidx_v1,
                 zbuf_v, data_v0, data_v1, tmp_v0, tmp_v1,
                 isem0, isem1, esem0, esem1, gsem0, gsem1):
    i32 = jnp.int32
    c = lax.axis_index("c").astype(i32)
    s = lax.axis_index("s").astype(i32)
    src_v = (src_v0, src_v1)
    dst_v = (dst_v0, dst_v1)
    eidx_v = (eidx_v0, eidx_v1)
    data_v = (data_v0, data_v1)
    tmp_v = (tmp_v0, tmp_v1)
    isem = (isem0, isem1)
    esem = (esem0, esem1)
    gsem = (gsem0, gsem1)
    iota = lax.iota(jnp.int32, 16)

    def _zero(i, _):
        for j in range(D // 16):
            zbuf_v[i, pl.ds(j * 16, 16)] = jnp.zeros((16,), jnp.float32)
        return 0
    lax.fori_loop(jnp.int32(0), jnp.int32(32), _zero, 0)
    z0 = s * i32(625)
    for off in range(0, 608, 32):
        pltpu.sync_copy(zbuf_v, acc.at[pl.ds(z0 + i32(off), 32)])
    pltpu.sync_copy(zbuf_v.at[pl.ds(0, 17)], acc.at[pl.ds(z0 + i32(608), 17)])
    plsc.subcore_barrier()

    def _base(k):
        k = jnp.minimum(k, i32(NCHE - 1))
        return pl.multiple_of(c * i32(EPC) + s * i32(EPSUB) + k * i32(CHE), 8)

    def load_idx(k, sl):
        b = _base(k)
        pltpu.async_copy(src_ref.at[pl.ds(b, CHE)], src_v[sl], isem[sl])
        pltpu.async_copy(dst_ref.at[pl.ds(b, CHE)], dst_v[sl], isem[sl])

    def issue_gathers(k, sl):
        b = _base(k)
        pltpu.make_async_copy(src_ref.at[pl.ds(b, CHE)], src_v[sl],
                              isem[sl]).wait()
        pltpu.make_async_copy(dst_ref.at[pl.ds(b, CHE)], dst_v[sl],
                              isem[sl]).wait()

        def _fill(j, _):
            eidx_v[sl][pl.ds(j * 16, 16)] = iota + b + j * 16
            return 0
        lax.fori_loop(jnp.int32(0), jnp.int32(CHE // 16), _fill, 0)
        pltpu.async_copy(xe_ref.at[eidx_v[sl]], data_v[sl], esem[sl])
        pltpu.async_copy(xn_ref.at[src_v[sl]], tmp_v[sl], gsem[sl])

    def process(k, sl):
        pltpu.make_async_copy(xe_ref.at[eidx_v[sl]], data_v[sl],
                              esem[sl]).wait()
        pltpu.make_async_copy(xn_ref.at[src_v[sl]], tmp_v[sl],
                              gsem[sl]).wait()
        pltpu.sync_copy(data_v[sl], acc.at[dst_v[sl]], add=True)
        pltpu.sync_copy(tmp_v[sl], acc.at[dst_v[sl]], add=True)

    load_idx(jnp.int32(0), 0)
    load_idx(jnp.int32(1), 1)
    issue_gathers(jnp.int32(0), 0)

    def _pipe(m, _):
        k0 = m * i32(2)
        issue_gathers(k0 + 1, 1)
        load_idx(k0 + 2, 0)
        process(k0, 0)
        issue_gathers(k0 + 2, 0)
        process(k0 + 1, 1)

        @pl.when(m < i32(NCHE // 2 - 1))
        def _():
            load_idx(k0 + 3, 1)
        return 0
    lax.fori_loop(jnp.int32(0), jnp.int32(NCHE // 2), _pipe, 0)
    process(jnp.int32(NCHE - 1), 0)

    plsc.subcore_barrier()

    @pl.when(s < 10)
    def _writeout():
        r0 = pl.multiple_of(s * i32(1000), 8)
        pltpu.sync_copy(acc.at[pl.ds(r0, 1000)],
                        part_ref.at[c, pl.ds(r0, 1000)])


def _sc_agg(xe, src, dst, xn):
    mesh = plsc.VectorSubcoreMesh(core_axis_name="c", subcore_axis_name="s")
    f = pl.kernel(
        _sc_agg_body,
        out_type=jax.ShapeDtypeStruct((NC, N, D), jnp.float32),
        mesh=mesh,
        scratch_types=[
            pltpu.VMEM_SHARED((N, D), jnp.float32),
            pltpu.VMEM((CHE,), jnp.int32),
            pltpu.VMEM((CHE,), jnp.int32),
            pltpu.VMEM((CHE,), jnp.int32),
            pltpu.VMEM((CHE,), jnp.int32),
            pltpu.VMEM((CHE,), jnp.int32),
            pltpu.VMEM((CHE,), jnp.int32),
            pltpu.VMEM((32, D), jnp.float32),
            pltpu.VMEM((CHE, D), jnp.float32),
            pltpu.VMEM((CHE, D), jnp.float32),
            pltpu.VMEM((CHE, D), jnp.float32),
            pltpu.VMEM((CHE, D), jnp.float32),
            pltpu.SemaphoreType.DMA,
            pltpu.SemaphoreType.DMA,
            pltpu.SemaphoreType.DMA,
            pltpu.SemaphoreType.DMA,
            pltpu.SemaphoreType.DMA,
            pltpu.SemaphoreType.DMA,
        ],
    )
    return f(xe, src, dst, xn)


def _pool_body(idx_ref, x_ref, o_ref, acc, cnt):
    step = pl.program_id(0)

    @pl.when(step == 0)
    def _init():
        acc[...] = jnp.zeros_like(acc)
        cnt[...] = jnp.zeros_like(cnt)

    idx = idx_ref[0, 0, :]
    onehot = (jax.lax.broadcasted_iota(jnp.int32, (G, _POOL_R), 0)
              == idx[None, :]).astype(jnp.float32)
    acc[...] += jnp.dot(onehot, x_ref[...],
                        preferred_element_type=jnp.float32)
    cnt[...] += jnp.sum(onehot, axis=1, keepdims=True)

    @pl.when(step == pl.num_programs(0) - 1)
    def _fini():
        o_ref[...] = acc[...] / jnp.clip(cnt[...], 1.0, None)


def _pool(x, idx):
    xp = jnp.zeros((_N_PAD, D), jnp.float32).at[:N].set(x)
    ip = jnp.full((_N_PAD,), G, jnp.int32).at[:N].set(idx.astype(jnp.int32))
    ip = ip.reshape(_N_PAD // _POOL_R, 1, _POOL_R)
    grid = _N_PAD // _POOL_R
    return pl.pallas_call(
        _pool_body,
        grid=(grid,),
        in_specs=[
            pl.BlockSpec((1, 1, _POOL_R), lambda i: (i, i * 0, i * 0)),
            pl.BlockSpec((_POOL_R, D), lambda i: (i, i * 0)),
        ],
        out_specs=pl.BlockSpec((G, D), lambda i: (i * 0, i * 0)),
        out_shape=jax.ShapeDtypeStruct((G, D), jnp.float32),
        scratch_shapes=[
            pltpu.VMEM((G, D), jnp.float32),
            pltpu.VMEM((G, 1), jnp.float32),
        ],
    )(ip, xp)


@functools.partial(jax.jit, static_argnums=())
def kernel(node, edge, n2n, e2n, idx_node, idx_edge, W_en, b_en, W_ee, b_ee,
           Wn, bn, We, be, W_fc, b_fc):
    del n2n, idx_edge, W_fc, b_fc
    src = e2n[0].astype(jnp.int32)
    dst = e2n[1].astype(jnp.int32)
    xn = node @ W_en + b_en
    xe = edge @ W_ee + b_ee
    L = Wn.shape[0]
    for i in range(L):
        part = _sc_agg(xe, src, dst, xn)
        agg = part[0] + part[1]
        xn = jax.nn.relu((xn + agg) @ Wn[i] + bn[i])
        if i < L - 1:
            xe = jax.nn.relu(
                (xe + jnp.take(xn, src, axis=0) + jnp.take(xn, dst, axis=0))
                @ We[i] + be[i])
    return _pool(xn, idx_node)


# speedup vs baseline: 2.1201x; 1.0049x over previous
import functools

import jax
import jax.numpy as jnp
from jax import lax
from jax.experimental import pallas as pl
from jax.experimental.pallas import tpu as pltpu
from jax.experimental.pallas import tpu_sc as plsc

N = 10000
E = 320000
G = 256
D = 128
DH = 64
NC = 2
NS = 16
EPS = E // NS
CH = 400
NCHUNK = EPS // CH
ROWS_PS = N // NS

_POOL_R = 1024
_N_PAD = 10240


EPC = E // NC
EPSUB = EPC // NS
CHE = 80
NCHE = EPSUB // CHE


def _sc_agg_body(xe_ref, src_ref, dst_ref, xn_ref, part_ref,
                 acc, src_v0, src_v1, dst_v0, dst_v1, eidx_v0, eidx_v1,
                 zbuf_v, data_v0, data_v1, tmp_v0, tmp_v1,
                 isem0, isem1, esem0, esem1, gsem0, gsem1):
    i32 = jnp.int32
    c = lax.axis_index("c").astype(i32)
    s = lax.axis_index("s").astype(i32)
    src_v = (src_v0, src_v1)
    dst_v = (dst_v0, dst_v1)
    eidx_v = (eidx_v0, eidx_v1)
    data_v = (data_v0, data_v1)
    tmp_v = (tmp_v0, tmp_v1)
    isem = (isem0, isem1)
    esem = (esem0, esem1)
    gsem = (gsem0, gsem1)
    iota = lax.iota(jnp.int32, 16)

    def _zero(i, _):
        for j in range(D // 16):
            zbuf_v[i, pl.ds(j * 16, 16)] = jnp.zeros((16,), jnp.float32)
        return 0
    lax.fori_loop(jnp.int32(0), jnp.int32(32), _zero, 0)
    z0 = s * i32(625)
    for off in range(0, 608, 32):
        pltpu.sync_copy(zbuf_v, acc.at[pl.ds(z0 + i32(off), 32)])
    pltpu.sync_copy(zbuf_v.at[pl.ds(0, 17)], acc.at[pl.ds(z0 + i32(608), 17)])
    plsc.subcore_barrier()

    def _base(k):
        k = jnp.minimum(k, i32(NCHE - 1))
        return pl.multiple_of(c * i32(EPC) + s * i32(EPSUB) + k * i32(CHE), 8)

    def load_idx(k, sl):
        b = _base(k)
        pltpu.async_copy(src_ref.at[pl.ds(b, CHE)], src_v[sl], isem[sl])
        pltpu.async_copy(dst_ref.at[pl.ds(b, CHE)], dst_v[sl], isem[sl])

    def issue_gathers(k, sl):
        b = _base(k)
        pltpu.make_async_copy(src_ref.at[pl.ds(b, CHE)], src_v[sl],
                              isem[sl]).wait()
        pltpu.make_async_copy(dst_ref.at[pl.ds(b, CHE)], dst_v[sl],
                              isem[sl]).wait()

        def _fill(j, _):
            eidx_v[sl][pl.ds(j * 16, 16)] = iota + b + j * 16
            return 0
        lax.fori_loop(jnp.int32(0), jnp.int32(CHE // 16), _fill, 0)
        pltpu.async_copy(xe_ref.at[eidx_v[sl]], data_v[sl], esem[sl])
        pltpu.async_copy(xn_ref.at[src_v[sl]], tmp_v[sl], gsem[sl])

    def process(k, sl):
        pltpu.make_async_copy(xe_ref.at[eidx_v[sl]], data_v[sl],
                              esem[sl]).wait()
        pltpu.make_async_copy(xn_ref.at[src_v[sl]], tmp_v[sl],
                              gsem[sl]).wait()
        pltpu.sync_copy(data_v[sl], acc.at[dst_v[sl]], add=True)
        pltpu.sync_copy(tmp_v[sl], acc.at[dst_v[sl]], add=True)

    load_idx(jnp.int32(0), 0)
    load_idx(jnp.int32(1), 1)
    issue_gathers(jnp.int32(0), 0)

    def _pipe(m, _):
        k0 = m * i32(2)
        issue_gathers(k0 + 1, 1)
        process(k0, 0)
        load_idx(k0 + 2, 0)
        issue_gathers(k0 + 2, 0)
        process(k0 + 1, 1)

        @pl.when(m < i32(NCHE // 2 - 1))
        def _():
            load_idx(k0 + 3, 1)
        return 0
    lax.fori_loop(jnp.int32(0), jnp.int32(NCHE // 2), _pipe, 0)
    process(jnp.int32(NCHE - 1), 0)

    plsc.subcore_barrier()

    @pl.when(s < 10)
    def _writeout():
        r0 = pl.multiple_of(s * i32(1000), 8)
        pltpu.sync_copy(acc.at[pl.ds(r0, 1000)],
                        part_ref.at[c, pl.ds(r0, 1000)])


def _sc_agg(xe, src, dst, xn):
    mesh = plsc.VectorSubcoreMesh(core_axis_name="c", subcore_axis_name="s")
    f = pl.kernel(
        _sc_agg_body,
        out_type=jax.ShapeDtypeStruct((NC, N, D), jnp.float32),
        mesh=mesh,
        scratch_types=[
            pltpu.VMEM_SHARED((N, D), jnp.float32),
            pltpu.VMEM((CHE,), jnp.int32),
            pltpu.VMEM((CHE,), jnp.int32),
            pltpu.VMEM((CHE,), jnp.int32),
            pltpu.VMEM((CHE,), jnp.int32),
            pltpu.VMEM((CHE,), jnp.int32),
            pltpu.VMEM((CHE,), jnp.int32),
            pltpu.VMEM((32, D), jnp.float32),
            pltpu.VMEM((CHE, D), jnp.float32),
            pltpu.VMEM((CHE, D), jnp.float32),
            pltpu.VMEM((CHE, D), jnp.float32),
            pltpu.VMEM((CHE, D), jnp.float32),
            pltpu.SemaphoreType.DMA,
            pltpu.SemaphoreType.DMA,
            pltpu.SemaphoreType.DMA,
            pltpu.SemaphoreType.DMA,
            pltpu.SemaphoreType.DMA,
            pltpu.SemaphoreType.DMA,
        ],
    )
    return f(xe, src, dst, xn)


def _pool_body(idx_ref, x_ref, o_ref, acc, cnt):
    step = pl.program_id(0)

    @pl.when(step == 0)
    def _init():
        acc[...] = jnp.zeros_like(acc)
        cnt[...] = jnp.zeros_like(cnt)

    idx = idx_ref[0, 0, :]
    onehot = (jax.lax.broadcasted_iota(jnp.int32, (G, _POOL_R), 0)
              == idx[None, :]).astype(jnp.float32)
    acc[...] += jnp.dot(onehot, x_ref[...],
                        preferred_element_type=jnp.float32)
    cnt[...] += jnp.sum(onehot, axis=1, keepdims=True)

    @pl.when(step == pl.num_programs(0) - 1)
    def _fini():
        o_ref[...] = acc[...] / jnp.clip(cnt[...], 1.0, None)


def _pool(x, idx):
    xp = jnp.zeros((_N_PAD, D), jnp.float32).at[:N].set(x)
    ip = jnp.full((_N_PAD,), G, jnp.int32).at[:N].set(idx.astype(jnp.int32))
    ip = ip.reshape(_N_PAD // _POOL_R, 1, _POOL_R)
    grid = _N_PAD // _POOL_R
    return pl.pallas_call(
        _pool_body,
        grid=(grid,),
        in_specs=[
            pl.BlockSpec((1, 1, _POOL_R), lambda i: (i, i * 0, i * 0)),
            pl.BlockSpec((_POOL_R, D), lambda i: (i, i * 0)),
        ],
        out_specs=pl.BlockSpec((G, D), lambda i: (i * 0, i * 0)),
        out_shape=jax.ShapeDtypeStruct((G, D), jnp.float32),
        scratch_shapes=[
            pltpu.VMEM((G, D), jnp.float32),
            pltpu.VMEM((G, 1), jnp.float32),
        ],
    )(ip, xp)


@functools.partial(jax.jit, static_argnums=())
def kernel(node, edge, n2n, e2n, idx_node, idx_edge, W_en, b_en, W_ee, b_ee,
           Wn, bn, We, be, W_fc, b_fc):
    del n2n, idx_edge, W_fc, b_fc
    src = e2n[0].astype(jnp.int32)
    dst = e2n[1].astype(jnp.int32)
    xn = node @ W_en + b_en
    xe = edge @ W_ee + b_ee
    L = Wn.shape[0]
    for i in range(L):
        part = _sc_agg(xe, src, dst, xn)
        agg = part[0] + part[1]
        xn = jax.nn.relu((xn + agg) @ Wn[i] + bn[i])
        if i < L - 1:
            xe = jax.nn.relu(
                (xe + jnp.take(xn, src, axis=0) + jnp.take(xn, dst, axis=0))
                @ We[i] + be[i])
    return _pool(xn, idx_node)
